# Initial kernel scaffold; baseline (speedup 1.0000x reference)
import jax
import jax.numpy as jnp
from jax.experimental import pallas as pl


def kernel(V_matrix, P_matrix, raw_base_points):
    raise NotImplementedError("write your pallas kernel here")


# speedup vs baseline: 8.0064x; 8.0064x over previous
import functools

import numpy as np
import jax
import jax.numpy as jnp
from jax import lax
from jax.experimental import pallas as pl
from jax.experimental.pallas import tpu as pltpu
from jax.experimental.pallas import tpu_sc as plsc

B = 64
N = 13860
H = W = 512
NPAD = 14336
NB = 2048
HW = H * W
NQ = 4
QSIZE = HW // NQ
SENT = -(1 << 20)
NC, NS = 2, 16
NWORK = NC * NS
IMGS_PER_WORKER = B // NWORK

def _bf16_round(x):
    u = lax.bitcast_convert_type(x, jnp.uint32)
    r = (u + jnp.uint32(0x7FFF) + ((u >> 16) & jnp.uint32(1))) & jnp.uint32(0xFFFF0000)
    return lax.bitcast_convert_type(r, jnp.float32)


def _proj_body(v_ref, p_ref, pts_ref, idx_ref):
    j = pl.program_id(0)
    Vm = _bf16_round(v_ref[:])
    Pm = _bf16_round(p_ref[:])
    rows = []
    for i in (0, 1, 3):
        acc = None
        for k in range(4):
            term = Pm[:, 4 * i + k:4 * i + k + 1] * Vm[:, 4 * k:4 * k + 4]
            acc = term if acc is None else acc + term
        rows.append(_bf16_round(acc))
    vpx, vpy, vpw = rows
    px = _bf16_round(pts_ref[0:1, :])
    py = _bf16_round(pts_ref[1:2, :])
    pz = _bf16_round(pts_ref[2:3, :])

    def proj(c):
        return c[:, 0:1] * px + c[:, 1:2] * py + c[:, 2:3] * pz + c[:, 3:4]

    tx = proj(vpx)
    ty = proj(vpy)
    tw = proj(vpw)
    nz = tw != 0.0
    X = jnp.where(nz, tx / tw, tx)
    Y = jnp.where(nz, ty / tw, ty)
    xs = (X + 1.0) * 0.5 * float(W)
    ys = (1.0 - (Y + 1.0) * 0.5) * float(H)
    xr = jnp.round(xs)
    yr = jnp.round(ys)
    valid = (xr >= 0.0) & (xr <= float(W - 1)) & (yr >= 0.0) & (yr <= float(H - 1))
    col = j * NB + lax.broadcasted_iota(jnp.int32, (B, NB), 1)
    valid = valid & (col < N)
    flat = yr.astype(jnp.int32) * W + xr.astype(jnp.int32)
    idx_ref[:] = jnp.where(valid, flat, SENT)


_proj_call = pl.pallas_call(
    _proj_body,
    grid=(NPAD // NB,),
    in_specs=[
        pl.BlockSpec((B, 16), lambda j: (0, 0)),
        pl.BlockSpec((B, 16), lambda j: (0, 0)),
        pl.BlockSpec((8, NB), lambda j: (0, j)),
    ],
    out_specs=pl.BlockSpec((B, NB), lambda j: (0, j)),
    out_shape=jax.ShapeDtypeStruct((B, NPAD), jnp.int32),
)


def _sc_scatter_body(idx_hbm, out_hbm, idx_v, qbuf):
    wid = lax.axis_index("s") * NC + lax.axis_index("c")
    zeros16 = jnp.zeros((16,), jnp.float32)
    v255 = jnp.full((16,), 255.0, jnp.float32)
    for im in range(IMGS_PER_WORKER):
        b = wid * IMGS_PER_WORKER + im
        pltpu.sync_copy(idx_hbm.at[b], idx_v)
        for q in range(NQ):
            lo = q * QSIZE

            @pl.loop(0, QSIZE // 16)
            def zero_body(i):
                qbuf[pl.ds(i * 16, 16)] = zeros16

            @pl.loop(0, NPAD // 16)
            def scan_body(i, lo=lo):
                v = idx_v[pl.ds(i * 16, 16)]
                m = (v >= lo) & (v < lo + QSIZE)
                loc = jnp.where(m, v - lo, 0)
                plsc.store_scatter(qbuf, [loc], v255, mask=m)

            pltpu.sync_copy(qbuf, out_hbm.at[b * NQ + q])


@functools.lru_cache(maxsize=1)
def _sc_scatter_call():
    mesh = plsc.VectorSubcoreMesh(
        core_axis_name="c", subcore_axis_name="s",
        num_cores=NC, num_subcores=NS)
    return pl.kernel(
        _sc_scatter_body,
        out_type=jax.ShapeDtypeStruct((B * NQ, QSIZE), jnp.float32),
        mesh=mesh,
        compiler_params=pltpu.CompilerParams(needs_layout_passes=False),
        scratch_types=[
            pltpu.VMEM((NPAD,), jnp.int32),
            pltpu.VMEM((QSIZE,), jnp.float32),
        ],
    )


def _gauss_weights():
    sigma = 0.3 * ((9 - 1) * 0.5 - 1.0) + 0.8
    x = np.arange(9, dtype=np.float32) - 4.0
    k = np.exp(-(x.astype(np.float32) ** 2) / np.float32(2.0 * sigma * sigma))
    k = k.astype(np.float32)
    k = k / k.sum(dtype=np.float32)
    return [float(v) for v in k]


_GW = _gauss_weights()


def _morph_body(r_ref, o_ref):
    r = r_ref[0]
    zc = jnp.zeros((H, 4), jnp.float32)
    phc = jnp.concatenate([zc, r, zc], axis=1)
    d1 = phc[:, 0:W]
    for k in range(1, 9):
        d1 = jnp.maximum(d1, phc[:, k:k + W])
    zr = jnp.zeros((4, W), jnp.float32)
    pv = jnp.concatenate([zr, d1, zr], axis=0)
    d = pv[0:H]
    for k in range(1, 9):
        d = jnp.maximum(d, pv[k:k + H])
    pvg = jnp.concatenate(
        [d[4:5], d[3:4], d[2:3], d[1:2], d,
         d[H - 2:H - 1], d[H - 3:H - 2], d[H - 4:H - 3], d[H - 5:H - 4]],
        axis=0)
    g0 = pvg[0:H] * _GW[0]
    for k in range(1, 9):
        g0 = g0 + pvg[k:k + H] * _GW[k]
    phg = jnp.concatenate(
        [g0[:, 4:5], g0[:, 3:4], g0[:, 2:3], g0[:, 1:2], g0,
         g0[:, W - 2:W - 1], g0[:, W - 3:W - 2],
         g0[:, W - 4:W - 3], g0[:, W - 5:W - 4]],
        axis=1)
    s = phg[:, 0:W] * _GW[0]
    for k in range(1, 9):
        s = s + phg[:, k:k + W] * _GW[k]
    ob = jnp.where(s > 100.0, 1.0, 0.0)
    o_ref[0, 0] = ob
    o_ref[0, 1] = ob
    o_ref[0, 2] = ob


_morph_call = pl.pallas_call(
    _morph_body,
    grid=(B,),
    in_specs=[pl.BlockSpec((1, H, W), lambda b: (b, 0, 0))],
    out_specs=pl.BlockSpec((1, 3, H, W), lambda b: (b, 0, 0, 0)),
    out_shape=jax.ShapeDtypeStruct((B, 3, H, W), jnp.float32),
)


def kernel(V_matrix, P_matrix, raw_base_points):
    V16 = V_matrix.reshape(B, 16)
    P16 = P_matrix.reshape(B, 16)
    ptsT = jnp.zeros((8, NPAD), jnp.float32)
    ptsT = ptsT.at[0:3, 0:N].set(raw_base_points[:, 0:3].T)
    idx = _proj_call(V16, P16, ptsT)
    raster = _sc_scatter_call()(idx)
    img = _morph_call(raster.reshape(B, H, W))
    return img


# speedup vs baseline: 15.3351x; 1.9153x over previous
import functools

import numpy as np
import jax
import jax.numpy as jnp
from jax import lax
from jax.experimental import pallas as pl
from jax.experimental.pallas import tpu as pltpu
from jax.experimental.pallas import tpu_sc as plsc

B = 64
N = 13860
H = W = 512
NPAD = 14336
NB = 2048
HW = H * W
NQ = 4
QSIZE = HW // NQ
SENT = -(1 << 20)
NC, NS = 2, 16
NWORK = NC * NS
IMGS_PER_WORKER = B // NWORK

def _bf16_round(x):
    u = lax.bitcast_convert_type(x, jnp.uint32)
    r = (u + jnp.uint32(0x7FFF) + ((u >> 16) & jnp.uint32(1))) & jnp.uint32(0xFFFF0000)
    return lax.bitcast_convert_type(r, jnp.float32)


def _proj_body(v_ref, p_ref, pts_ref, idx_ref):
    j = pl.program_id(0)
    Vm = _bf16_round(v_ref[:])
    Pm = _bf16_round(p_ref[:])
    rows = []
    for i in (0, 1, 3):
        acc = None
        for k in range(4):
            term = Pm[:, 4 * i + k:4 * i + k + 1] * Vm[:, 4 * k:4 * k + 4]
            acc = term if acc is None else acc + term
        rows.append(_bf16_round(acc))
    vpx, vpy, vpw = rows
    px = _bf16_round(pts_ref[0:1, :])
    py = _bf16_round(pts_ref[1:2, :])
    pz = _bf16_round(pts_ref[2:3, :])

    def proj(c):
        return c[:, 0:1] * px + c[:, 1:2] * py + c[:, 2:3] * pz + c[:, 3:4]

    tx = proj(vpx)
    ty = proj(vpy)
    tw = proj(vpw)
    nz = tw != 0.0
    X = jnp.where(nz, tx / tw, tx)
    Y = jnp.where(nz, ty / tw, ty)
    xs = (X + 1.0) * 0.5 * float(W)
    ys = (1.0 - (Y + 1.0) * 0.5) * float(H)
    xr = jnp.round(xs)
    yr = jnp.round(ys)
    valid = (xr >= 0.0) & (xr <= float(W - 1)) & (yr >= 0.0) & (yr <= float(H - 1))
    col = j * NB + lax.broadcasted_iota(jnp.int32, (B, NB), 1)
    valid = valid & (col < N)
    flat = yr.astype(jnp.int32) * W + xr.astype(jnp.int32)
    idx_ref[:] = jnp.where(valid, flat, SENT)


_proj_call = pl.pallas_call(
    _proj_body,
    grid=(NPAD // NB,),
    in_specs=[
        pl.BlockSpec((B, 16), lambda j: (0, 0)),
        pl.BlockSpec((B, 16), lambda j: (0, 0)),
        pl.BlockSpec((8, NB), lambda j: (0, j)),
    ],
    out_specs=pl.BlockSpec((B, NB), lambda j: (0, j)),
    out_shape=jax.ShapeDtypeStruct((B, NPAD), jnp.int32),
)


def _sc_scatter_body(idx_hbm, out_hbm, idx_v, qbuf):
    wid = lax.axis_index("s") * NC + lax.axis_index("c")
    zeros16 = jnp.zeros((16,), jnp.float32)
    v255 = jnp.full((16,), 255.0, jnp.float32)
    for im in range(IMGS_PER_WORKER):
        b = wid * IMGS_PER_WORKER + im
        pltpu.sync_copy(idx_hbm.at[b], idx_v)
        for q in range(NQ):
            lo = q * QSIZE

            @pl.loop(0, QSIZE // 16, unroll=8)
            def zero_body(i):
                qbuf[pl.ds(i * 16, 16)] = zeros16

            @pl.loop(0, NPAD // 16, unroll=4)
            def scan_body(i, lo=lo):
                v = idx_v[pl.ds(i * 16, 16)]
                m = (v >= lo) & (v < lo + QSIZE)
                loc = jnp.where(m, v - lo, 0)
                plsc.store_scatter(qbuf, [loc], v255, mask=m)

            pltpu.sync_copy(qbuf, out_hbm.at[b * NQ + q])


@functools.lru_cache(maxsize=1)
def _sc_scatter_call():
    mesh = plsc.VectorSubcoreMesh(
        core_axis_name="c", subcore_axis_name="s",
        num_cores=NC, num_subcores=NS)
    return pl.kernel(
        _sc_scatter_body,
        out_type=jax.ShapeDtypeStruct((B * NQ, QSIZE), jnp.float32),
        mesh=mesh,
        compiler_params=pltpu.CompilerParams(needs_layout_passes=False),
        scratch_types=[
            pltpu.VMEM((NPAD,), jnp.int32),
            pltpu.VMEM((QSIZE,), jnp.float32),
        ],
    )


def _gauss_weights():
    sigma = 0.3 * ((9 - 1) * 0.5 - 1.0) + 0.8
    x = np.arange(9, dtype=np.float32) - 4.0
    k = np.exp(-(x.astype(np.float32) ** 2) / np.float32(2.0 * sigma * sigma))
    k = k.astype(np.float32)
    k = k / k.sum(dtype=np.float32)
    return [float(v) for v in k]


_GW = _gauss_weights()


def _band_ones():
    i = np.arange(H)
    return (np.abs(i[:, None] - i[None, :]) <= 4).astype(np.float32)


def _gauss_op():
    g = np.zeros((H, H), np.float32)
    gw = np.asarray(_GW, np.float32)
    for i in range(H):
        for k in range(9):
            t = i + k - 4
            if t < 0:
                t = -t
            elif t > H - 1:
                t = 2 * (H - 1) - t
            g[i, t] += gw[k]
    return g


def _morph_body(bo_ref, gg_ref, ggt_ref, r_ref, o_ref):
    m = r_ref[0]
    bo = bo_ref[:]
    c = jnp.dot(bo, m)
    c2 = jnp.dot(c, bo)
    d = jnp.where(c2 > 0.5, 255.0, 0.0)
    hp = jax.lax.Precision.HIGHEST
    s1 = jnp.dot(gg_ref[:], d, precision=hp)
    s = jnp.dot(s1, ggt_ref[:], precision=hp)
    ob = jnp.where(s > 100.0, 1.0, 0.0)
    o_ref[0, 0] = ob
    o_ref[0, 1] = ob
    o_ref[0, 2] = ob


_morph_call = pl.pallas_call(
    _morph_body,
    grid=(B,),
    in_specs=[
        pl.BlockSpec((H, H), lambda b: (0, 0)),
        pl.BlockSpec((H, H), lambda b: (0, 0)),
        pl.BlockSpec((H, H), lambda b: (0, 0)),
        pl.BlockSpec((1, H, W), lambda b: (b, 0, 0)),
    ],
    out_specs=pl.BlockSpec((1, 3, H, W), lambda b: (b, 0, 0, 0)),
    out_shape=jax.ShapeDtypeStruct((B, 3, H, W), jnp.float32),
)


def kernel(V_matrix, P_matrix, raw_base_points):
    V16 = V_matrix.reshape(B, 16)
    P16 = P_matrix.reshape(B, 16)
    ptsT = jnp.zeros((8, NPAD), jnp.float32)
    ptsT = ptsT.at[0:3, 0:N].set(raw_base_points[:, 0:3].T)
    idx = _proj_call(V16, P16, ptsT)
    raster = _sc_scatter_call()(idx)
    bo = jnp.asarray(_band_ones())
    gg = jnp.asarray(_gauss_op())
    img = _morph_call(bo, gg, gg.T, raster.reshape(B, H, W))
    return img


# speedup vs baseline: 16.8690x; 1.1000x over previous
import functools

import numpy as np
import jax
import jax.numpy as jnp
from jax import lax
from jax.experimental import pallas as pl
from jax.experimental.pallas import tpu as pltpu
from jax.experimental.pallas import tpu_sc as plsc

B = 64
N = 13860
H = W = 512
NPAD = 14336
NB = 2048
HW = H * W
NQ = 4
QSIZE = HW // NQ
SENT = -(1 << 20)
NC, NS = 2, 16
NWORK = NC * NS
IMGS_PER_WORKER = B // NWORK

def _bf16_round(x):
    u = lax.bitcast_convert_type(x, jnp.uint32)
    r = (u + jnp.uint32(0x7FFF) + ((u >> 16) & jnp.uint32(1))) & jnp.uint32(0xFFFF0000)
    return lax.bitcast_convert_type(r, jnp.float32)


def _proj_body(v_ref, p_ref, pts_ref, idx_ref):
    j = pl.program_id(0)
    Vm = _bf16_round(v_ref[:])
    Pm = _bf16_round(p_ref[:])
    rows = []
    for i in (0, 1, 3):
        acc = None
        for k in range(4):
            term = Pm[:, 4 * i + k:4 * i + k + 1] * Vm[:, 4 * k:4 * k + 4]
            acc = term if acc is None else acc + term
        rows.append(_bf16_round(acc))
    vpx, vpy, vpw = rows
    px = _bf16_round(pts_ref[0:1, :])
    py = _bf16_round(pts_ref[1:2, :])
    pz = _bf16_round(pts_ref[2:3, :])

    def proj(c):
        return c[:, 0:1] * px + c[:, 1:2] * py + c[:, 2:3] * pz + c[:, 3:4]

    tx = proj(vpx)
    ty = proj(vpy)
    tw = proj(vpw)
    nz = tw != 0.0
    X = jnp.where(nz, tx / tw, tx)
    Y = jnp.where(nz, ty / tw, ty)
    xs = (X + 1.0) * 0.5 * float(W)
    ys = (1.0 - (Y + 1.0) * 0.5) * float(H)
    xr = jnp.round(xs)
    yr = jnp.round(ys)
    valid = (xr >= 0.0) & (xr <= float(W - 1)) & (yr >= 0.0) & (yr <= float(H - 1))
    col = j * NB + lax.broadcasted_iota(jnp.int32, (B, NB), 1)
    valid = valid & (col < N)
    flat = yr.astype(jnp.int32) * W + xr.astype(jnp.int32)
    idx_ref[:] = jnp.where(valid, flat, SENT)


_proj_call = pl.pallas_call(
    _proj_body,
    grid=(NPAD // NB,),
    in_specs=[
        pl.BlockSpec((B, 16), lambda j: (0, 0)),
        pl.BlockSpec((B, 16), lambda j: (0, 0)),
        pl.BlockSpec((8, NB), lambda j: (0, j)),
    ],
    out_specs=pl.BlockSpec((B, NB), lambda j: (0, j)),
    out_shape=jax.ShapeDtypeStruct((B, NPAD), jnp.int32),
)


QROWS = H // NQ


def _sc_scatter_body(idx_hbm, out_hbm, idx_v, qbuf):
    wid = lax.axis_index("s") * NC + lax.axis_index("c")
    zeros16 = jnp.zeros((16,), jnp.float32)
    v255 = jnp.full((16,), 255.0, jnp.float32)
    for im in range(IMGS_PER_WORKER):
        b = wid * IMGS_PER_WORKER + im
        pltpu.sync_copy(idx_hbm.at[b], idx_v)
        for q in range(NQ):
            lo = q * QSIZE

            @pl.loop(0, QROWS)
            def zero_body(i):
                for cc in range(W // 16):
                    qbuf[i, pl.ds(cc * 16, 16)] = zeros16

            @pl.loop(0, NPAD // 16, unroll=8)
            def scan_body(i, lo=lo):
                v = idx_v[pl.ds(i * 16, 16)]
                m = (v >= lo) & (v < lo + QSIZE)
                vq = jnp.where(m, v - lo, 0)
                row = vq >> 9
                col = vq & (W - 1)
                plsc.store_scatter(qbuf, [row, col], v255, mask=m)

            pltpu.sync_copy(qbuf, out_hbm.at[b, pl.ds(q * QROWS, QROWS)])


@functools.lru_cache(maxsize=1)
def _sc_scatter_call():
    mesh = plsc.VectorSubcoreMesh(
        core_axis_name="c", subcore_axis_name="s",
        num_cores=NC, num_subcores=NS)
    return pl.kernel(
        _sc_scatter_body,
        out_type=jax.ShapeDtypeStruct((B, H, W), jnp.float32),
        mesh=mesh,
        compiler_params=pltpu.CompilerParams(needs_layout_passes=False),
        scratch_types=[
            pltpu.VMEM((NPAD,), jnp.int32),
            pltpu.VMEM((QROWS, W), jnp.float32),
        ],
    )


def _gauss_weights():
    sigma = 0.3 * ((9 - 1) * 0.5 - 1.0) + 0.8
    x = np.arange(9, dtype=np.float32) - 4.0
    k = np.exp(-(x.astype(np.float32) ** 2) / np.float32(2.0 * sigma * sigma))
    k = k.astype(np.float32)
    k = k / k.sum(dtype=np.float32)
    return [float(v) for v in k]


_GW = _gauss_weights()


def _band_ones():
    i = np.arange(H)
    return (np.abs(i[:, None] - i[None, :]) <= 4).astype(np.float32)


def _gauss_op():
    g = np.zeros((H, H), np.float32)
    gw = np.asarray(_GW, np.float32)
    for i in range(H):
        for k in range(9):
            t = i + k - 4
            if t < 0:
                t = -t
            elif t > H - 1:
                t = 2 * (H - 1) - t
            g[i, t] += gw[k]
    return g


def _morph_body(bo_ref, gg_ref, ggt_ref, r_ref, o_ref):
    m = r_ref[0]
    bo = bo_ref[:]
    c = jnp.dot(bo, m)
    c2 = jnp.dot(c, bo)
    d = jnp.where(c2 > 0.5, 255.0, 0.0)
    hp = jax.lax.Precision.HIGHEST
    s1 = jnp.dot(gg_ref[:], d, precision=hp)
    s = jnp.dot(s1, ggt_ref[:], precision=hp)
    ob = jnp.where(s > 100.0, 1.0, 0.0)
    o_ref[0, 0] = ob
    o_ref[0, 1] = ob
    o_ref[0, 2] = ob


_morph_call = pl.pallas_call(
    _morph_body,
    grid=(B,),
    in_specs=[
        pl.BlockSpec((H, H), lambda b: (0, 0)),
        pl.BlockSpec((H, H), lambda b: (0, 0)),
        pl.BlockSpec((H, H), lambda b: (0, 0)),
        pl.BlockSpec((1, H, W), lambda b: (b, 0, 0)),
    ],
    out_specs=pl.BlockSpec((1, 3, H, W), lambda b: (b, 0, 0, 0)),
    out_shape=jax.ShapeDtypeStruct((B, 3, H, W), jnp.float32),
)


def kernel(V_matrix, P_matrix, raw_base_points):
    V16 = V_matrix.reshape(B, 16)
    P16 = P_matrix.reshape(B, 16)
    ptsT = jnp.zeros((8, NPAD), jnp.float32)
    ptsT = ptsT.at[0:3, 0:N].set(raw_base_points[:, 0:3].T)
    idx = _proj_call(V16, P16, ptsT)
    raster = _sc_scatter_call()(idx)
    bo = jnp.asarray(_band_ones())
    gg = jnp.asarray(_gauss_op())
    img = _morph_call(bo, gg, gg.T, raster)
    return img
